# Initial kernel scaffold; baseline (speedup 1.0000x reference)
import jax
import jax.numpy as jnp
from jax.experimental import pallas as pl


def kernel(x, w1, b1, w2, b2):
    raise NotImplementedError("write your pallas kernel here")


# speedup vs baseline: 23.1287x; 23.1287x over previous
import functools

import jax
import jax.numpy as jnp
from jax.experimental import pallas as pl
from jax.experimental.pallas import tpu as pltpu

_BN_EPS = 1e-5


def _bn_mlp_body(n_tiles, tile_b, batch, masked,
                 xT_ref, w1_ref, b1_ref, w2_ref, b2_ref,
                 oT_ref, h_ref, sum_ref, ssq_ref):
    p = pl.program_id(0)
    t = pl.program_id(1)
    inv_b = 1.0 / float(batch)

    @pl.when(p == 0)
    def _stats_pass():
        @pl.when(t == 0)
        def _():
            sum_ref[...] = jnp.zeros_like(sum_ref)
            ssq_ref[...] = jnp.zeros_like(ssq_ref)

        h = (jnp.dot(w1_ref[...], xT_ref[...],
                     preferred_element_type=jnp.float32)
             + b1_ref[...])
        h_ref[t] = h
        if masked:
            col = t * tile_b + jax.lax.broadcasted_iota(jnp.int32, h.shape, 1)
            h = jnp.where(col < batch, h, 0.0)
        sum_ref[...] += jnp.sum(h, axis=1, keepdims=True)
        ssq_ref[...] += jnp.sum(h * h, axis=1, keepdims=True)

    @pl.when(p == 1)
    def _output_pass():
        mean = sum_ref[...] * inv_b
        var = jnp.maximum(ssq_ref[...] * inv_b - mean * mean, 0.0)
        scale = jax.lax.rsqrt(var + _BN_EPS)
        shift = -mean * scale
        hn = jnp.maximum(h_ref[t] * scale + shift, 0.0)
        y = jnp.sum(hn * w2_ref[...], axis=0, keepdims=True) + b2_ref[0, 0]
        oT_ref[...] = jax.nn.sigmoid(y)


@functools.partial(jax.jit, static_argnames=())
def kernel(x, w1, b1, w2, b2):
    batch = x.shape[0]
    hid = w1.shape[0]
    xT = x.T

    tile_b = 65536
    if batch % tile_b != 0:
        tile_b = 8192 if batch % 8192 == 0 else 128
    n_tiles = -(-batch // tile_b)
    padded = n_tiles * tile_b
    masked = padded != batch
    if masked:
        xT = jnp.pad(xT, ((0, 0), (0, padded - batch)))

    body = functools.partial(_bn_mlp_body, n_tiles, tile_b, batch, masked)

    yT = pl.pallas_call(
        body,
        out_shape=jax.ShapeDtypeStruct((1, padded), jnp.float32),
        grid=(2, n_tiles),
        in_specs=[
            pl.BlockSpec((hid, tile_b),
                         lambda p, t: (0, t * (1 - p) + (n_tiles - 1) * p)),
            pl.BlockSpec((hid, hid), lambda p, t: (0, 0)),
            pl.BlockSpec((hid, 1), lambda p, t: (0, 0)),
            pl.BlockSpec((hid, 1), lambda p, t: (0, 0)),
            pl.BlockSpec(memory_space=pltpu.MemorySpace.SMEM),
        ],
        out_specs=pl.BlockSpec((1, tile_b), lambda p, t: (0, t * p)),
        scratch_shapes=[
            pltpu.VMEM((n_tiles, hid, tile_b), jnp.float32),
            pltpu.VMEM((hid, 1), jnp.float32),
            pltpu.VMEM((hid, 1), jnp.float32),
        ],
        compiler_params=pltpu.CompilerParams(
            dimension_semantics=("arbitrary", "arbitrary"),
            vmem_limit_bytes=52 * 1024 * 1024,
        ),
    )(xT, w1, b1, w2, b2)

    return yT[:, :batch].reshape(batch, 1)


# speedup vs baseline: 33.6725x; 1.4559x over previous
import functools

import jax
import jax.numpy as jnp
from jax.experimental import pallas as pl
from jax.experimental.pallas import tpu as pltpu

_BN_EPS = 1e-5
_CHUNKS = 8


def _bn_mlp_body(tile_b, tile_c, batch, masked,
                 xT_ref, w1_ref, w2blk_ref, b2_ref,
                 o_ref, h_ref, sum_ref, ssq_ref):
    i = pl.program_id(0)
    inv_b = 1.0 / float(batch)

    @pl.when(i < _CHUNKS)
    def _stats_pass():
        @pl.when(i == 0)
        def _():
            sum_ref[...] = jnp.zeros_like(sum_ref)
            ssq_ref[...] = jnp.zeros_like(ssq_ref)

        h = jnp.dot(w1_ref[...], xT_ref[...],
                    preferred_element_type=jnp.float32)
        h_ref[i] = h
        if masked:
            col = i * tile_b + jax.lax.broadcasted_iota(jnp.int32, h.shape, 1)
            h = jnp.where(col < batch, h, 0.0)
        sum_ref[...] += jnp.sum(h, axis=1, keepdims=True)
        ssq_ref[...] += jnp.sum(h * h, axis=1, keepdims=True)

    @pl.when(i >= _CHUNKS)
    def _output_pass():
        g = i - _CHUNKS
        mean = sum_ref[...] * inv_b
        var = jnp.maximum(ssq_ref[...] * inv_b - mean * mean, 0.0)
        scale = jax.lax.rsqrt(var + _BN_EPS)
        shift = -mean * scale
        hn = jnp.concatenate(
            [jnp.maximum(
                h_ref[s, :, pl.ds(g * tile_c, tile_c)] * scale + shift, 0.0)
             for s in range(_CHUNKS)], axis=0)
        y = jnp.dot(w2blk_ref[...], hn,
                    preferred_element_type=jnp.float32) + b2_ref[0, 0]
        o_ref[...] = jax.nn.sigmoid(y)


def kernel(x, w1, b1, w2, b2):
    del b1
    batch = x.shape[0]
    hid = w1.shape[0]
    xT = x.T

    padded = -(-batch // (_CHUNKS * 128)) * (_CHUNKS * 128)
    masked = padded != batch
    if masked:
        xT = jnp.pad(xT, ((0, 0), (0, padded - batch)))
    tile_b = padded // _CHUNKS
    n_out = max(1, tile_b // 16384)
    if tile_b % n_out:
        n_out = 1
    tile_c = tile_b // n_out

    w2blk = jnp.kron(jnp.eye(_CHUNKS, dtype=jnp.float32), w2.reshape(1, hid))

    body = functools.partial(_bn_mlp_body, tile_b, tile_c, batch, masked)
    last = _CHUNKS - 1

    out = pl.pallas_call(
        body,
        out_shape=jax.ShapeDtypeStruct((_CHUNKS, padded // _CHUNKS),
                                       jnp.float32),
        grid=(_CHUNKS + n_out,),
        in_specs=[
            pl.BlockSpec((hid, tile_b), lambda i: (0, jnp.minimum(i, last))),
            pl.BlockSpec((hid, hid), lambda i: (0, 0)),
            pl.BlockSpec((_CHUNKS, _CHUNKS * hid), lambda i: (0, 0)),
            pl.BlockSpec(memory_space=pltpu.MemorySpace.SMEM),
        ],
        out_specs=pl.BlockSpec((_CHUNKS, tile_c),
                               lambda i: (0, jnp.maximum(i - _CHUNKS, 0))),
        scratch_shapes=[
            pltpu.VMEM((_CHUNKS, hid, tile_b), jnp.float32),
            pltpu.VMEM((hid, 1), jnp.float32),
            pltpu.VMEM((hid, 1), jnp.float32),
        ],
        compiler_params=pltpu.CompilerParams(
            dimension_semantics=("arbitrary",),
            vmem_limit_bytes=52 * 1024 * 1024,
        ),
    )(xT, w1, w2blk, b2)

    return out.reshape(padded, 1)[:batch]
